# baseline (speedup 1.0000x reference)
import functools

import jax
import jax.numpy as jnp
from jax import lax
from jax.experimental import pallas as pl
from jax.experimental.pallas import tpu as pltpu
from jax.experimental.pallas import tpu_sc as plsc

_BETA = 10.0
_N = 1_000_000
_NC = 2
_NS = 16
_NW = _NC * _NS
_STRIDE = 31_248
_W = 31_312
_VECS = _W // 16
_C0 = 979
_C1 = _VECS - _C0
_E0 = _C0 * 16
_E1 = _C1 * 16


def _gumbel_noise_numpy():
    import numpy as np

    def rotl(x, r):
        return ((x << np.uint32(r)) | (x >> np.uint32(32 - r))).astype(np.uint32)

    ks0, ks1 = np.uint32(0), np.uint32(42)
    ks2 = np.uint32(ks0 ^ ks1 ^ np.uint32(0x1BD11BDA))
    x0 = np.full(_N, ks0, np.uint32)
    x1 = (np.arange(_N, dtype=np.uint32) + ks1).astype(np.uint32)
    rotations = [(13, 15, 26, 6), (17, 29, 16, 24)]
    ks = [ks0, ks1, ks2]
    for i in range(5):
        for r in rotations[i % 2]:
            x0 = (x0 + x1).astype(np.uint32)
            x1 = (rotl(x1, r) ^ x0).astype(np.uint32)
        x0 = (x0 + ks[(i + 1) % 3]).astype(np.uint32)
        x1 = (x1 + ks[(i + 2) % 3] + np.uint32(i + 1)).astype(np.uint32)
    bits = (x0 ^ x1).astype(np.uint32)
    mant = (bits >> np.uint32(9)) | np.uint32(0x3F800000)
    u = mant.view(np.float32) - np.float32(1.0)
    tiny = np.float32(np.finfo(np.float32).tiny)
    u = np.maximum(tiny, (u * (np.float32(1.0) - tiny) + tiny).astype(np.float32))
    return (-np.log(-np.log(u.astype(np.float64)))).astype(np.float32)


try:
    _G = jax.jit(lambda: jax.random.gumbel(jax.random.key(42), (_N,), jnp.float32))()
    _G.block_until_ready()
except Exception:
    _G = _gumbel_noise_numpy()

_mesh = plsc.VectorSubcoreMesh(core_axis_name="c", subcore_axis_name="s")


@functools.partial(
    pl.kernel,
    out_type=(
        jax.ShapeDtypeStruct((_NW * 16,), jnp.float32),
        jax.ShapeDtypeStruct((_NW * 16,), jnp.int32),
    ),
    mesh=_mesh,
    scratch_types=(
        pltpu.VMEM((_E0,), jnp.float32),
        pltpu.VMEM((_E0,), jnp.float32),
        pltpu.VMEM((_E1,), jnp.float32),
        pltpu.VMEM((_E1,), jnp.float32),
        pltpu.VMEM((16,), jnp.float32),
        pltpu.VMEM((16,), jnp.int32),
        pltpu.SemaphoreType.DMA,
        pltpu.SemaphoreType.DMA,
        pltpu.SemaphoreType.DMA,
        pltpu.SemaphoreType.DMA,
    ),
)
def _sc_partial_argmax(scores_hbm, g_hbm, outv_hbm, outi_hbm,
                       s0_v, g0_v, s1_v, g1_v, mv, mi,
                       sem_s0, sem_g0, sem_s1, sem_g1):
    wid = lax.axis_index("s") * _NC + lax.axis_index("c")
    base = wid * _STRIDE
    cp_s0 = pltpu.async_copy(scores_hbm.at[pl.ds(base, _E0)], s0_v, sem_s0)
    cp_g0 = pltpu.async_copy(g_hbm.at[pl.ds(base, _E0)], g0_v, sem_g0)
    cp_s1 = pltpu.async_copy(scores_hbm.at[pl.ds(base + _E0, _E1)], s1_v, sem_s1)
    cp_g1 = pltpu.async_copy(g_hbm.at[pl.ds(base + _E0, _E1)], g1_v, sem_g1)

    def make_body(s_v, g_v):
        def body(i, carry):
            vmax, vit = carry
            off = i * 16
            z = s_v[pl.ds(off, 16)] * jnp.float32(_BETA) + g_v[pl.ds(off, 16)]
            take = z > vmax
            return jnp.where(take, z, vmax), jnp.where(take, jnp.full((16,), 0, jnp.int32) + i, vit)
        return body

    init = (jnp.full((16,), -jnp.inf, jnp.float32), jnp.zeros((16,), jnp.int32))
    cp_s0.wait()
    cp_g0.wait()
    vmax, vit0 = lax.fori_loop(0, _C0, make_body(s0_v, g0_v), init, unroll=4)
    cp_s1.wait()
    cp_g1.wait()
    vmax, vit = lax.fori_loop(0, _C1, make_body(s1_v, g1_v),
                              (vmax, vit0 - _C0), unroll=4)
    lane = lax.iota(jnp.int32, 16)
    vidx = base + (vit + _C0) * 16 + lane

    mv[...] = vmax
    mi[...] = vidx
    pltpu.sync_copy(mv, outv_hbm.at[pl.ds(wid * 16, 16)])
    pltpu.sync_copy(mi, outi_hbm.at[pl.ds(wid * 16, 16)])


def _merge_body(v_ref, i_ref, o_ref):
    v = v_ref[...]
    ii = i_ref[...]
    m = jnp.max(v)
    big = jnp.where(v == m, ii, jnp.int32(2147483647))
    o_ref[0] = jnp.min(big)


def kernel(scores):
    vals, idxs = _sc_partial_argmax(scores, jnp.asarray(_G))
    merged = pl.pallas_call(
        _merge_body,
        out_shape=jax.ShapeDtypeStruct((1,), jnp.int32),
        out_specs=pl.BlockSpec(memory_space=pltpu.SMEM),
    )(vals.reshape(4, 128), idxs.reshape(4, 128))
    return merged[0]


# speedup vs baseline: 1.0168x; 1.0168x over previous
import functools

import jax
import jax.numpy as jnp
from jax import lax
from jax.experimental import pallas as pl
from jax.experimental.pallas import tpu as pltpu
from jax.experimental.pallas import tpu_sc as plsc

_BETA = 10.0
_N = 1_000_000
_NC = 2
_NS = 16
_NW = _NC * _NS
_STRIDE = 31_248
_W = 31_312
_VECS = _W // 16
_C0 = 979
_C1 = _VECS - _C0
_E0 = _C0 * 16
_E1 = _C1 * 16


def _gumbel_noise_numpy():
    import numpy as np

    def rotl(x, r):
        return ((x << np.uint32(r)) | (x >> np.uint32(32 - r))).astype(np.uint32)

    ks0, ks1 = np.uint32(0), np.uint32(42)
    ks2 = np.uint32(ks0 ^ ks1 ^ np.uint32(0x1BD11BDA))
    x0 = np.full(_N, ks0, np.uint32)
    x1 = (np.arange(_N, dtype=np.uint32) + ks1).astype(np.uint32)
    rotations = [(13, 15, 26, 6), (17, 29, 16, 24)]
    ks = [ks0, ks1, ks2]
    for i in range(5):
        for r in rotations[i % 2]:
            x0 = (x0 + x1).astype(np.uint32)
            x1 = (rotl(x1, r) ^ x0).astype(np.uint32)
        x0 = (x0 + ks[(i + 1) % 3]).astype(np.uint32)
        x1 = (x1 + ks[(i + 2) % 3] + np.uint32(i + 1)).astype(np.uint32)
    bits = (x0 ^ x1).astype(np.uint32)
    mant = (bits >> np.uint32(9)) | np.uint32(0x3F800000)
    u = mant.view(np.float32) - np.float32(1.0)
    tiny = np.float32(np.finfo(np.float32).tiny)
    u = np.maximum(tiny, (u * (np.float32(1.0) - tiny) + tiny).astype(np.float32))
    return (-np.log(-np.log(u.astype(np.float64)))).astype(np.float32)


try:
    _G = jax.jit(lambda: jax.random.gumbel(jax.random.key(42), (_N,), jnp.float32))()
    _G.block_until_ready()
except Exception:
    _G = _gumbel_noise_numpy()

_mesh = plsc.VectorSubcoreMesh(core_axis_name="c", subcore_axis_name="s")


@functools.partial(
    pl.kernel,
    out_type=(
        jax.ShapeDtypeStruct((_NW * 16,), jnp.float32),
        jax.ShapeDtypeStruct((_NW * 16,), jnp.int32),
    ),
    mesh=_mesh,
    scratch_types=(
        pltpu.VMEM((_W,), jnp.float32),
        pltpu.VMEM((_W,), jnp.float32),
        pltpu.VMEM((16,), jnp.float32),
        pltpu.VMEM((16,), jnp.int32),
        pltpu.SemaphoreType.DMA,
        pltpu.SemaphoreType.DMA,
    ),
)
def _sc_partial_argmax(scores_hbm, g_hbm, outv_hbm, outi_hbm,
                       s_v, g_v, mv, mi, sem_s, sem_g):
    wid = lax.axis_index("s") * _NC + lax.axis_index("c")
    base = wid * _STRIDE
    cp_s = pltpu.async_copy(scores_hbm.at[pl.ds(base, _W)], s_v, sem_s)
    cp_g = pltpu.async_copy(g_hbm.at[pl.ds(base, _W)], g_v, sem_g)
    cp_s.wait()
    cp_g.wait()

    def body(i, carry):
        vmax, vit = carry
        off = i * 16
        z = s_v[pl.ds(off, 16)] * jnp.float32(_BETA) + g_v[pl.ds(off, 16)]
        take = z > vmax
        return (jnp.where(take, z, vmax),
                jnp.where(take, jnp.full((16,), 0, jnp.int32) + i, vit))

    init = (jnp.full((16,), -jnp.inf, jnp.float32), jnp.zeros((16,), jnp.int32))
    vmax, vit = lax.fori_loop(0, _VECS, body, init, unroll=4)
    lane = lax.iota(jnp.int32, 16)
    vidx = base + vit * 16 + lane

    mv[...] = vmax
    mi[...] = vidx
    cp_v = pltpu.async_copy(mv, outv_hbm.at[pl.ds(wid * 16, 16)], sem_s)
    cp_i = pltpu.async_copy(mi, outi_hbm.at[pl.ds(wid * 16, 16)], sem_g)
    cp_v.wait()
    cp_i.wait()


def _merge_body(v_ref, i_ref, o_ref):
    v = v_ref[...]
    ii = i_ref[...]
    m = jnp.max(v)
    big = jnp.where(v == m, ii, jnp.int32(2147483647))
    o_ref[0] = jnp.min(big)


def kernel(scores):
    vals, idxs = _sc_partial_argmax(scores, jnp.asarray(_G))
    merged = pl.pallas_call(
        _merge_body,
        out_shape=jax.ShapeDtypeStruct((1,), jnp.int32),
        out_specs=pl.BlockSpec(memory_space=pltpu.SMEM),
    )(vals.reshape(4, 128), idxs.reshape(4, 128))
    return merged[0]


# speedup vs baseline: 1.9829x; 1.9502x over previous
import functools

import jax
import jax.numpy as jnp
from jax import lax
from jax.experimental import pallas as pl
from jax.experimental.pallas import tpu as pltpu

_BETA = 10.0
_N = 1_000_000
_ROWS = 64
_LANES = 15_625
_GRID = 8
_BR = _ROWS // _GRID


def _gumbel_noise_numpy():
    import numpy as np

    def rotl(x, r):
        return ((x << np.uint32(r)) | (x >> np.uint32(32 - r))).astype(np.uint32)

    ks0, ks1 = np.uint32(0), np.uint32(42)
    ks2 = np.uint32(ks0 ^ ks1 ^ np.uint32(0x1BD11BDA))
    x0 = np.full(_N, ks0, np.uint32)
    x1 = (np.arange(_N, dtype=np.uint32) + ks1).astype(np.uint32)
    rotations = [(13, 15, 26, 6), (17, 29, 16, 24)]
    ks = [ks0, ks1, ks2]
    for i in range(5):
        for r in rotations[i % 2]:
            x0 = (x0 + x1).astype(np.uint32)
            x1 = (rotl(x1, r) ^ x0).astype(np.uint32)
        x0 = (x0 + ks[(i + 1) % 3]).astype(np.uint32)
        x1 = (x1 + ks[(i + 2) % 3] + np.uint32(i + 1)).astype(np.uint32)
    bits = (x0 ^ x1).astype(np.uint32)
    mant = (bits >> np.uint32(9)) | np.uint32(0x3F800000)
    u = mant.view(np.float32) - np.float32(1.0)
    tiny = np.float32(np.finfo(np.float32).tiny)
    u = np.maximum(tiny, (u * (np.float32(1.0) - tiny) + tiny).astype(np.float32))
    return (-np.log(-np.log(u.astype(np.float64)))).astype(np.float32)


try:
    _G = jax.jit(lambda: jax.random.gumbel(jax.random.key(42), (_N,), jnp.float32))()
    _G.block_until_ready()
except Exception:
    _G = _gumbel_noise_numpy()


def _argmax_body(s_ref, g_ref, o_ref, vmax, vidx):
    pid = pl.program_id(0)
    s = s_ref[...]
    g = g_ref[...]
    z = s * jnp.float32(_BETA) + g
    row = lax.broadcasted_iota(jnp.int32, (_BR, _LANES), 0)
    col = lax.broadcasted_iota(jnp.int32, (_BR, _LANES), 1)
    idx = (pid * (_BR * _LANES)) + row * _LANES + col

    @pl.when(pid == 0)
    def _init():
        vmax[...] = z
        vidx[...] = idx

    @pl.when(pid != 0)
    def _acc():
        take = z > vmax[...]
        vmax[...] = jnp.where(take, z, vmax[...])
        vidx[...] = jnp.where(take, idx, vidx[...])

    @pl.when(pid == _GRID - 1)
    def _finish():
        m = jnp.max(vmax[...])
        big = jnp.where(vmax[...] == m, vidx[...], jnp.int32(2147483647))
        o_ref[0] = jnp.min(big)


@functools.partial(jax.jit, static_argnums=())
def _run(scores2, g2):
    return pl.pallas_call(
        _argmax_body,
        grid=(_GRID,),
        in_specs=[
            pl.BlockSpec((_BR, _LANES), lambda i: (i, 0)),
            pl.BlockSpec((_BR, _LANES), lambda i: (i, 0)),
        ],
        out_specs=pl.BlockSpec(memory_space=pltpu.SMEM),
        out_shape=jax.ShapeDtypeStruct((1,), jnp.int32),
        scratch_shapes=[
            pltpu.VMEM((_BR, _LANES), jnp.float32),
            pltpu.VMEM((_BR, _LANES), jnp.int32),
        ],
    )(scores2, g2)


def kernel(scores):
    scores2 = scores.reshape(_ROWS, _LANES)
    g2 = jnp.asarray(_G).reshape(_ROWS, _LANES)
    out = _run(scores2, g2)
    return out[0]
